# Initial kernel scaffold; baseline (speedup 1.0000x reference)
import functools

import jax
import jax.numpy as jnp
from jax import lax
from jax.experimental import pallas as pl
from jax.experimental.pallas import tpu as pltpu
from jax.experimental.pallas import tpu_sc as plsc

B = 8
N = 4096
KNB = 32
S1 = 2048
S2 = 1024
C1 = 64
C2 = 128
C3 = 256
EPS = 1e-5
BIGF = 3.0e38

F32 = jnp.float32


def _norm_consts(st, cnt, g, b):
    m = st[0:1, :] / cnt
    v = st[1:2, :] / cnt - m * m
    scale = g / jnp.sqrt(v + EPS)
    shift = b - m * scale
    return scale, shift


def _stats_contrib(y):
    s = jnp.sum(y, axis=0, keepdims=True)
    s2 = jnp.sum(y * y, axis=0, keepdims=True)
    pad = jnp.zeros((6, y.shape[-1]), F32)
    return jnp.concatenate([s, s2, pad], axis=0)


def _e1_body(x_ref, w_ref, y_ref, st_ref):
    i = pl.program_id(0)
    y = jnp.dot(x_ref[...], w_ref[...], preferred_element_type=F32)
    y_ref[...] = y
    contrib = _stats_contrib(y)

    @pl.when(i == 0)
    def _():
        st_ref[...] = jnp.zeros_like(st_ref)

    st_ref[...] += contrib


def _mlp_mid_body(cnt, y_ref, st_ref, g_ref, b_ref, w_ref, o_ref, st2_ref):
    i = pl.program_id(0)
    scale, shift = _norm_consts(st_ref[...], cnt, g_ref[...], b_ref[...])
    yn = jnp.maximum(y_ref[...] * scale + shift, 0.0)
    y2 = jnp.dot(yn, w_ref[...], preferred_element_type=F32)
    o_ref[...] = y2
    contrib = _stats_contrib(y2)

    @pl.when(i == 0)
    def _():
        st2_ref[...] = jnp.zeros_like(st2_ref)

    st2_ref[...] += contrib


def _tables_body(cnt, y_ref, st_ref, g_ref, b_ref, wn_ref, wd_ref, zn_ref, zd_ref):
    scale, shift = _norm_consts(st_ref[...], cnt, g_ref[...], b_ref[...])
    yn = jnp.maximum(y_ref[...] * scale + shift, 0.0)
    zn_ref[...] = jnp.dot(yn, wn_ref[...], preferred_element_type=F32)
    zd_ref[...] = jnp.dot(yn, wd_ref[...], preferred_element_type=F32)


def _embed(xp, w1p, g1, b1, w2t, g2, b2, wn1t, wd1t, interpret=False):
    R = B * N
    RB = 2048
    grid = (R // RB,)
    y1, st1 = pl.pallas_call(
        _e1_body,
        grid=grid,
        in_specs=[pl.BlockSpec((RB, 128), lambda i: (i, 0)),
                  pl.BlockSpec((128, C1), lambda i: (0, 0))],
        out_specs=[pl.BlockSpec((RB, C1), lambda i: (i, 0)),
                   pl.BlockSpec((8, C1), lambda i: (0, 0))],
        out_shape=[jax.ShapeDtypeStruct((R, C1), F32),
                   jax.ShapeDtypeStruct((8, C1), F32)],
        interpret=interpret,
    )(xp, w1p)
    y2, st2 = pl.pallas_call(
        functools.partial(_mlp_mid_body, float(R)),
        grid=grid,
        in_specs=[pl.BlockSpec((RB, C1), lambda i: (i, 0)),
                  pl.BlockSpec((8, C1), lambda i: (0, 0)),
                  pl.BlockSpec((1, C1), lambda i: (0, 0)),
                  pl.BlockSpec((1, C1), lambda i: (0, 0)),
                  pl.BlockSpec((C1, C1), lambda i: (0, 0))],
        out_specs=[pl.BlockSpec((RB, C1), lambda i: (i, 0)),
                   pl.BlockSpec((8, C1), lambda i: (0, 0))],
        out_shape=[jax.ShapeDtypeStruct((R, C1), F32),
                   jax.ShapeDtypeStruct((8, C1), F32)],
        interpret=interpret,
    )(y1, st1, g1, b1, w2t)
    zn, zd = pl.pallas_call(
        functools.partial(_tables_body, float(R)),
        grid=grid,
        in_specs=[pl.BlockSpec((RB, C1), lambda i: (i, 0)),
                  pl.BlockSpec((8, C1), lambda i: (0, 0)),
                  pl.BlockSpec((1, C1), lambda i: (0, 0)),
                  pl.BlockSpec((1, C1), lambda i: (0, 0)),
                  pl.BlockSpec((C1, C2), lambda i: (0, 0)),
                  pl.BlockSpec((C1, C2), lambda i: (0, 0))],
        out_specs=[pl.BlockSpec((RB, C2), lambda i: (i, 0)),
                   pl.BlockSpec((RB, C2), lambda i: (i, 0))],
        out_shape=[jax.ShapeDtypeStruct((R, C2), F32),
                   jax.ShapeDtypeStruct((R, C2), F32)],
        interpret=interpret,
    )(y2, st2, g2, b2, wn1t, wd1t)
    return zn, zd


def _fps_body(P, S, c_ref, idx_ref, sc_ref):
    px = c_ref[0]
    py = c_ref[1]
    pz = c_ref[2]
    boff = lax.broadcasted_iota(jnp.int32, (B, 1), 0) * P
    lane = lax.broadcasted_iota(jnp.int32, (B, P), 1)
    q0x = px[:, 0:1]
    q0y = py[:, 0:1]
    q0z = pz[:, 0:1]
    idx_ref[:, 0:1] = boff
    sc_ref[0, :, 0:1] = q0x
    sc_ref[1, :, 0:1] = q0y
    sc_ref[2, :, 0:1] = q0z
    dx = px - q0x
    dy = py - q0y
    dz = pz - q0z
    mind = (dx * dx + dy * dy) + dz * dz

    def body(i, mind):
        mx = jnp.max(mind, axis=1, keepdims=True)
        am = jnp.min(jnp.where(mind == mx, lane, P), axis=1, keepdims=True)
        onehot = lane == am
        qx = jnp.sum(jnp.where(onehot, px, 0.0), axis=1, keepdims=True)
        qy = jnp.sum(jnp.where(onehot, py, 0.0), axis=1, keepdims=True)
        qz = jnp.sum(jnp.where(onehot, pz, 0.0), axis=1, keepdims=True)
        idx_ref[:, pl.ds(i, 1)] = am + boff
        sc_ref[0, :, pl.ds(i, 1)] = qx
        sc_ref[1, :, pl.ds(i, 1)] = qy
        sc_ref[2, :, pl.ds(i, 1)] = qz
        ddx = px - qx
        ddy = py - qy
        ddz = pz - qz
        d = (ddx * ddx + ddy * ddy) + ddz * ddz
        return jnp.minimum(mind, d)

    lax.fori_loop(1, S, body, mind)


def _fps(coords_t, P, S, interpret=False):
    return pl.pallas_call(
        functools.partial(_fps_body, P, S),
        in_specs=[pl.BlockSpec((3, B, P), lambda: (0, 0, 0))],
        out_specs=[pl.BlockSpec((B, S), lambda: (0, 0)),
                   pl.BlockSpec((3, B, S), lambda: (0, 0, 0))],
        out_shape=[jax.ShapeDtypeStruct((B, S), jnp.int32),
                   jax.ShapeDtypeStruct((3, B, S), F32)],
        interpret=interpret,
    )(coords_t)


def _knn_body(P, T, q_ref, p_ref, o_ref):
    qx = q_ref[0][:, :, None]
    qy = q_ref[1][:, :, None]
    qz = q_ref[2][:, :, None]
    px = p_ref[0][:, None, :]
    py = p_ref[1][:, None, :]
    pz = p_ref[2][:, None, :]
    q2 = (qx * qx + qy * qy) + qz * qz
    p2 = (px * px + py * py) + pz * pz
    qp = qx * px + qy * py + qz * pz
    d = (q2 - 2.0 * qp) + p2
    lane = lax.broadcasted_iota(jnp.int32, (B, T, P), 2)
    boff = lax.broadcasted_iota(jnp.int32, (B, 1, 1), 0) * P
    picks = []
    for _ in range(KNB):
        mn = jnp.min(d, axis=2, keepdims=True)
        am = jnp.min(jnp.where(d == mn, lane, P), axis=2, keepdims=True)
        picks.append(am)
        d = jnp.where(lane == am, BIGF, d)
    o_ref[...] = jnp.concatenate(picks, axis=2) + boff


def _knn(q_t, p_t, P, S, interpret=False):
    T = 64
    return pl.pallas_call(
        functools.partial(_knn_body, P, T),
        grid=(S // T,),
        in_specs=[pl.BlockSpec((3, B, T), lambda i: (0, 0, i)),
                  pl.BlockSpec((3, B, P), lambda i: (0, 0, 0))],
        out_specs=pl.BlockSpec((B, T, KNB), lambda i: (0, i, 0)),
        out_shape=jax.ShapeDtypeStruct((B, S, KNB), jnp.int32),
        interpret=interpret,
    )(q_t, p_t)


_SC_INFO = plsc.get_sparse_core_info()
_NC = _SC_INFO.num_cores
_NS = _SC_INFO.num_subcores
_NW = _NC * _NS
_GCH = 128


def _make_gather(V, D, M):
    per_w = M // _NW
    nch = per_w // _GCH
    assert per_w % _GCH == 0 and M % _NW == 0
    mesh = plsc.VectorSubcoreMesh(core_axis_name="c", subcore_axis_name="s")

    @functools.partial(
        pl.kernel,
        mesh=mesh,
        out_type=jax.ShapeDtypeStruct((M, D), F32),
        scratch_types=[
            pltpu.VMEM((_GCH,), jnp.int32),
            pltpu.VMEM((_GCH, D), F32),
            pltpu.SemaphoreType.DMA,
        ],
    )
    def k(idx_hbm, table_hbm, out_hbm, idx_v, rows_v, sem):
        wid = lax.axis_index("s") * _NC + lax.axis_index("c")
        base = wid * per_w

        def body(i, carry):
            off = base + i * _GCH
            pltpu.sync_copy(idx_hbm.at[pl.ds(off, _GCH)], idx_v)
            pltpu.async_copy(table_hbm.at[idx_v], rows_v, sem).wait()
            pltpu.sync_copy(rows_v, out_hbm.at[pl.ds(off, _GCH)])
            return carry

        lax.fori_loop(0, nch, body, 0)

    return k


def _p1_body(QB, C, n_ref, vx_ref, st_ref):
    i = pl.program_id(0)
    y = n_ref[...].reshape(QB, KNB, C) + vx_ref[...][:, None, :]
    s = jnp.sum(y, axis=(0, 1))[None]
    s2 = jnp.sum(y * y, axis=(0, 1))[None]
    pad = jnp.zeros((6, C), F32)
    contrib = jnp.concatenate([s, s2, pad], axis=0)

    @pl.when(i == 0)
    def _():
        st_ref[...] = jnp.zeros_like(st_ref)

    st_ref[...] += contrib


def _p2_body(QB, C, cnt, n_ref, vx_ref, st_ref, g_ref, b_ref, w_ref, o_ref, st2_ref):
    i = pl.program_id(0)
    scale, shift = _norm_consts(st_ref[...], cnt, g_ref[...], b_ref[...])
    y = n_ref[...].reshape(QB, KNB, C) + vx_ref[...][:, None, :]
    yn = jnp.maximum(y * scale[None] + shift[None], 0.0).reshape(QB * KNB, C)
    y2 = jnp.dot(yn, w_ref[...], preferred_element_type=F32)
    o_ref[...] = y2
    contrib = _stats_contrib(y2)

    @pl.when(i == 0)
    def _():
        st2_ref[...] = jnp.zeros_like(st2_ref)

    st2_ref[...] += contrib


def _p3_tables_body(QB, C, cnt, y_ref, st_ref, g_ref, b_ref, wn_ref, wd_ref,
                    zn_ref, zd_ref):
    scale, shift = _norm_consts(st_ref[...], cnt, g_ref[...], b_ref[...])
    yn = jnp.maximum(y_ref[...] * scale + shift, 0.0).reshape(QB, KNB, C)
    pooled = jnp.max(yn, axis=1)
    zn_ref[...] = jnp.dot(pooled, wn_ref[...], preferred_element_type=F32)
    zd_ref[...] = jnp.dot(pooled, wd_ref[...], preferred_element_type=F32)


def _p3_final_body(QB, C, cnt, y_ref, st_ref, g_ref, b_ref, wo_ref, y3_ref, st3_ref):
    i = pl.program_id(0)
    scale, shift = _norm_consts(st_ref[...], cnt, g_ref[...], b_ref[...])
    yn = jnp.maximum(y_ref[...] * scale + shift, 0.0).reshape(QB, KNB, C)
    pooled = jnp.max(yn, axis=1)
    y3 = jnp.dot(pooled, wo_ref[...], preferred_element_type=F32)
    y3_ref[...] = y3
    contrib = _stats_contrib(y3)

    @pl.when(i == 0)
    def _():
        st3_ref[...] = jnp.zeros_like(st3_ref)

    st3_ref[...] += contrib


def _p5_body(y_ref, st_ref, g_ref, b_ref, o_ref):
    scale, shift = _norm_consts(st_ref[...], float(B * S2), g_ref[...], b_ref[...])
    o_ref[...] = (y_ref[...] * scale + shift)[None]


def _stage_mlp(n_rows, C, nflat, vxflat, ga, ba, wbt, interpret=False):
    RB = 4096
    QB = RB // KNB
    grid = (n_rows // RB,)
    cnt = float(n_rows)
    st_a = pl.pallas_call(
        functools.partial(_p1_body, QB, C),
        grid=grid,
        in_specs=[pl.BlockSpec((RB, C), lambda i: (i, 0)),
                  pl.BlockSpec((QB, C), lambda i: (i, 0))],
        out_specs=pl.BlockSpec((8, C), lambda i: (0, 0)),
        out_shape=jax.ShapeDtypeStruct((8, C), F32),
        interpret=interpret,
    )(nflat, vxflat)
    y2, st_b = pl.pallas_call(
        functools.partial(_p2_body, QB, C, cnt),
        grid=grid,
        in_specs=[pl.BlockSpec((RB, C), lambda i: (i, 0)),
                  pl.BlockSpec((QB, C), lambda i: (i, 0)),
                  pl.BlockSpec((8, C), lambda i: (0, 0)),
                  pl.BlockSpec((1, C), lambda i: (0, 0)),
                  pl.BlockSpec((1, C), lambda i: (0, 0)),
                  pl.BlockSpec((C, C), lambda i: (0, 0))],
        out_specs=[pl.BlockSpec((RB, C), lambda i: (i, 0)),
                   pl.BlockSpec((8, C), lambda i: (0, 0))],
        out_shape=[jax.ShapeDtypeStruct((n_rows, C), F32),
                   jax.ShapeDtypeStruct((8, C), F32)],
        interpret=interpret,
    )(nflat, vxflat, st_a, ga, ba, wbt)
    return y2, st_b


def kernel(x, w_emb1, gamma_emb1, beta_emb1, w_emb2, gamma_emb2, beta_emb2,
           w_sg1_a, gamma_sg1_a, beta_sg1_a, w_sg1_b, gamma_sg1_b, beta_sg1_b,
           w_sg2_a, gamma_sg2_a, beta_sg2_a, w_sg2_b, gamma_sg2_b, beta_sg2_b,
           w_out, gamma_out, beta_out):
    f = lambda a: a.astype(F32)
    x = f(x)
    xp = jnp.zeros((B * N, 128), F32).at[:, :3].set(x.reshape(B * N, 3))
    w1p = jnp.zeros((128, C1), F32).at[:3, :].set(f(w_emb1).T)
    coords_t = jnp.transpose(x, (2, 0, 1))
    r1 = lambda a: f(a).reshape(1, -1)
    wn1t = f(w_sg1_a)[:, :C1].T
    wd1t = (f(w_sg1_a)[:, C1:] - f(w_sg1_a)[:, :C1]).T
    wn2t = f(w_sg2_a)[:, :C2].T
    wd2t = (f(w_sg2_a)[:, C2:] - f(w_sg2_a)[:, :C2]).T

    zn1, zd1 = _embed(xp, w1p, r1(gamma_emb1), r1(beta_emb1), f(w_emb2).T,
                      r1(gamma_emb2), r1(beta_emb2), wn1t, wd1t)

    idx1, sct1 = _fps(coords_t, N, S1)
    knn1 = _knn(sct1, coords_t, N, S1)

    vx1 = _make_gather(B * N, C2, B * S1)(idx1.reshape(-1), zd1)
    n1 = _make_gather(B * N, C2, B * S1 * KNB)(knn1.reshape(-1), zn1)

    y2s1, stb1 = _stage_mlp(B * S1 * KNB, C2, n1, vx1,
                            r1(gamma_sg1_a), r1(beta_sg1_a), f(w_sg1_b).T)
    RB = 4096
    QB = RB // KNB
    zn2, zd2 = pl.pallas_call(
        functools.partial(_p3_tables_body, QB, C2, float(B * S1 * KNB)),
        grid=(B * S1 * KNB // RB,),
        in_specs=[pl.BlockSpec((RB, C2), lambda i: (i, 0)),
                  pl.BlockSpec((8, C2), lambda i: (0, 0)),
                  pl.BlockSpec((1, C2), lambda i: (0, 0)),
                  pl.BlockSpec((1, C2), lambda i: (0, 0)),
                  pl.BlockSpec((C2, C3), lambda i: (0, 0)),
                  pl.BlockSpec((C2, C3), lambda i: (0, 0))],
        out_specs=[pl.BlockSpec((QB, C3), lambda i: (i, 0)),
                   pl.BlockSpec((QB, C3), lambda i: (i, 0))],
        out_shape=[jax.ShapeDtypeStruct((B * S1, C3), F32),
                   jax.ShapeDtypeStruct((B * S1, C3), F32)],
    )(y2s1, stb1, r1(gamma_sg1_b), r1(beta_sg1_b), wn2t, wd2t)

    idx2, sct2 = _fps(sct1, S1, S2)
    knn2 = _knn(sct2, sct1, S1, S2)

    vx2 = _make_gather(B * S1, C3, B * S2)(idx2.reshape(-1), zd2)
    n2 = _make_gather(B * S1, C3, B * S2 * KNB)(knn2.reshape(-1), zn2)

    y2s2, stb2 = _stage_mlp(B * S2 * KNB, C3, n2, vx2,
                            r1(gamma_sg2_a), r1(beta_sg2_a), f(w_sg2_b).T)
    y3, st3 = pl.pallas_call(
        functools.partial(_p3_final_body, QB, C3, float(B * S2 * KNB)),
        grid=(B * S2 * KNB // RB,),
        in_specs=[pl.BlockSpec((RB, C3), lambda i: (i, 0)),
                  pl.BlockSpec((8, C3), lambda i: (0, 0)),
                  pl.BlockSpec((1, C3), lambda i: (0, 0)),
                  pl.BlockSpec((1, C3), lambda i: (0, 0)),
                  pl.BlockSpec((C3, C3), lambda i: (0, 0))],
        out_specs=[pl.BlockSpec((QB, C3), lambda i: (i, 0)),
                   pl.BlockSpec((8, C3), lambda i: (0, 0))],
        out_shape=[jax.ShapeDtypeStruct((B * S2, C3), F32),
                   jax.ShapeDtypeStruct((8, C3), F32)],
    )(y2s2, stb2, r1(gamma_sg2_b), r1(beta_sg2_b), f(w_out).T)
    hfin = pl.pallas_call(
        _p5_body,
        grid=(B,),
        in_specs=[pl.BlockSpec((S2, C3), lambda i: (i, 0)),
                  pl.BlockSpec((8, C3), lambda i: (0, 0)),
                  pl.BlockSpec((1, C3), lambda i: (0, 0)),
                  pl.BlockSpec((1, C3), lambda i: (0, 0))],
        out_specs=pl.BlockSpec((1, S2, C3), lambda i: (i, 0, 0)),
        out_shape=jax.ShapeDtypeStruct((B, S2, C3), F32),
    )(y3, st3, r1(gamma_out), r1(beta_out))

    h_out = jnp.transpose(hfin, (0, 2, 1))
    coords_out = jnp.transpose(sct2, (1, 2, 0))
    return h_out, coords_out


# speedup vs baseline: 11.7483x; 11.7483x over previous
import functools

import jax
import jax.numpy as jnp
from jax import lax
from jax.experimental import pallas as pl
from jax.experimental.pallas import tpu as pltpu
from jax.experimental.pallas import tpu_sc as plsc

B = 8
N = 4096
KNB = 32
S1 = 2048
S2 = 1024
C1 = 64
C2 = 128
C3 = 256
EPS = 1e-5
BIGF = 3.0e38

F32 = jnp.float32


def _norm_consts(st, cnt, g, b):
    m = st[0:1, :] / cnt
    v = st[1:2, :] / cnt - m * m
    scale = g / jnp.sqrt(v + EPS)
    shift = b - m * scale
    return scale, shift


def _bf(a):
    return a.astype(jnp.bfloat16)


def _bdot(a, b):
    return jnp.dot(_bf(a), _bf(b), preferred_element_type=F32)


def _stats_contrib(y):
    s = jnp.sum(y, axis=0, keepdims=True)
    s2 = jnp.sum(y * y, axis=0, keepdims=True)
    pad = jnp.zeros((6, y.shape[-1]), F32)
    return jnp.concatenate([s, s2, pad], axis=0)


def _e1_body(x_ref, w_ref, y_ref, st_ref):
    i = pl.program_id(0)
    y = _bdot(x_ref[...], w_ref[...])
    y_ref[...] = y
    contrib = _stats_contrib(y)

    @pl.when(i == 0)
    def _():
        st_ref[...] = jnp.zeros_like(st_ref)

    st_ref[...] += contrib


def _mlp_mid_body(cnt, y_ref, st_ref, g_ref, b_ref, w_ref, o_ref, st2_ref):
    i = pl.program_id(0)
    scale, shift = _norm_consts(st_ref[...], cnt, g_ref[...], b_ref[...])
    yn = jnp.maximum(y_ref[...] * scale + shift, 0.0)
    y2 = _bdot(yn, w_ref[...])
    o_ref[...] = y2
    contrib = _stats_contrib(y2)

    @pl.when(i == 0)
    def _():
        st2_ref[...] = jnp.zeros_like(st2_ref)

    st2_ref[...] += contrib


def _hout_body(cnt, y_ref, st_ref, g_ref, b_ref, h_ref):
    scale, shift = _norm_consts(st_ref[...], cnt, g_ref[...], b_ref[...])
    yn = jnp.maximum(y_ref[...] * scale + shift, 0.0)
    h_ref[...] = jnp.concatenate(
        [yn, jnp.zeros((yn.shape[0], 128 - yn.shape[1]), F32)], axis=1)


def _embed(xp, w1p, g1, b1, w2t, g2, b2, interpret=False):
    R = B * N
    RB = 2048
    grid = (R // RB,)
    y1, st1 = pl.pallas_call(
        _e1_body,
        grid=grid,
        in_specs=[pl.BlockSpec((RB, 128), lambda i: (i, 0)),
                  pl.BlockSpec((128, C1), lambda i: (0, 0))],
        out_specs=[pl.BlockSpec((RB, C1), lambda i: (i, 0)),
                   pl.BlockSpec((8, C1), lambda i: (0, 0))],
        out_shape=[jax.ShapeDtypeStruct((R, C1), F32),
                   jax.ShapeDtypeStruct((8, C1), F32)],
        interpret=interpret,
    )(xp, w1p)
    y2, st2 = pl.pallas_call(
        functools.partial(_mlp_mid_body, float(R)),
        grid=grid,
        in_specs=[pl.BlockSpec((RB, C1), lambda i: (i, 0)),
                  pl.BlockSpec((8, C1), lambda i: (0, 0)),
                  pl.BlockSpec((1, C1), lambda i: (0, 0)),
                  pl.BlockSpec((1, C1), lambda i: (0, 0)),
                  pl.BlockSpec((C1, C1), lambda i: (0, 0))],
        out_specs=[pl.BlockSpec((RB, C1), lambda i: (i, 0)),
                   pl.BlockSpec((8, C1), lambda i: (0, 0))],
        out_shape=[jax.ShapeDtypeStruct((R, C1), F32),
                   jax.ShapeDtypeStruct((8, C1), F32)],
        interpret=interpret,
    )(y1, st1, g1, b1, w2t)
    h = pl.pallas_call(
        functools.partial(_hout_body, float(R)),
        grid=grid,
        in_specs=[pl.BlockSpec((RB, C1), lambda i: (i, 0)),
                  pl.BlockSpec((8, C1), lambda i: (0, 0)),
                  pl.BlockSpec((1, C1), lambda i: (0, 0)),
                  pl.BlockSpec((1, C1), lambda i: (0, 0))],
        out_specs=pl.BlockSpec((RB, 128), lambda i: (i, 0)),
        out_shape=jax.ShapeDtypeStruct((R, 128), F32),
        interpret=interpret,
    )(y2, st2, g2, b2)
    return h


def _fps_body(P, S, c_ref, idx_ref, sc_ref):
    px = c_ref[0]
    py = c_ref[1]
    pz = c_ref[2]
    boff = lax.broadcasted_iota(jnp.int32, (B, 1), 0) * P
    lane = lax.broadcasted_iota(jnp.int32, (B, P), 1)
    slane = lax.broadcasted_iota(jnp.int32, (B, S), 1)
    q0x = px[:, 0:1]
    q0y = py[:, 0:1]
    q0z = pz[:, 0:1]
    idx_acc = jnp.broadcast_to(boff, (B, S))
    qx_acc = jnp.broadcast_to(q0x, (B, S))
    qy_acc = jnp.broadcast_to(q0y, (B, S))
    qz_acc = jnp.broadcast_to(q0z, (B, S))
    dx = px - q0x
    dy = py - q0y
    dz = pz - q0z
    mind = (dx * dx + dy * dy) + dz * dz

    def body(i, carry):
        mind, idx_acc, qx_acc, qy_acc, qz_acc = carry
        mx = jnp.max(mind, axis=1, keepdims=True)
        am = jnp.min(jnp.where(mind == mx, lane, P), axis=1, keepdims=True)
        onehot = lane == am
        qx = jnp.sum(jnp.where(onehot, px, 0.0), axis=1, keepdims=True)
        qy = jnp.sum(jnp.where(onehot, py, 0.0), axis=1, keepdims=True)
        qz = jnp.sum(jnp.where(onehot, pz, 0.0), axis=1, keepdims=True)
        sel = slane == i
        idx_acc = jnp.where(sel, am + boff, idx_acc)
        qx_acc = jnp.where(sel, qx, qx_acc)
        qy_acc = jnp.where(sel, qy, qy_acc)
        qz_acc = jnp.where(sel, qz, qz_acc)
        ddx = px - qx
        ddy = py - qy
        ddz = pz - qz
        d = (ddx * ddx + ddz * ddz) + ddy * ddy
        return (jnp.minimum(mind, d), idx_acc, qx_acc, qy_acc, qz_acc)

    carry = (mind, idx_acc, qx_acc, qy_acc, qz_acc)
    _, idx_acc, qx_acc, qy_acc, qz_acc = lax.fori_loop(1, S, body, carry)
    idx_ref[...] = idx_acc
    sc_ref[0] = qx_acc
    sc_ref[1] = qy_acc
    sc_ref[2] = qz_acc


def _fps(coords_t, P, S, interpret=False):
    return pl.pallas_call(
        functools.partial(_fps_body, P, S),
        in_specs=[pl.BlockSpec((3, B, P), lambda: (0, 0, 0))],
        out_specs=[pl.BlockSpec((B, S), lambda: (0, 0)),
                   pl.BlockSpec((3, B, S), lambda: (0, 0, 0))],
        out_shape=[jax.ShapeDtypeStruct((B, S), jnp.int32),
                   jax.ShapeDtypeStruct((3, B, S), F32)],
        interpret=interpret,
    )(coords_t)


def _knn_body(P, T, q_ref, p_ref, o_ref):
    qx = q_ref[0][:, :, None]
    qy = q_ref[1][:, :, None]
    qz = q_ref[2][:, :, None]
    px = p_ref[0][:, None, :]
    py = p_ref[1][:, None, :]
    pz = p_ref[2][:, None, :]
    q2 = (qx * qx + qy * qy) + qz * qz
    p2 = (px * px + py * py) + pz * pz
    bf = lambda a: a.astype(jnp.bfloat16).astype(F32)
    qp = bf(qx) * bf(px) + bf(qy) * bf(py) + bf(qz) * bf(pz)
    d = (q2 - 2.0 * qp) + p2
    lane = lax.broadcasted_iota(jnp.int32, (B, T, P), 2)
    boff = lax.broadcasted_iota(jnp.int32, (B, 1, 1), 0) * P
    picks = []
    for _ in range(KNB):
        mn = jnp.min(d, axis=2, keepdims=True)
        am = jnp.min(jnp.where(d == mn, lane, P), axis=2, keepdims=True)
        picks.append(am)
        d = jnp.where(lane == am, BIGF, d)
    o_ref[...] = jnp.concatenate(picks, axis=2) + boff


def _knn(q_t, p_t, P, S, interpret=False):
    T = 128
    return pl.pallas_call(
        functools.partial(_knn_body, P, T),
        grid=(S // T,),
        in_specs=[pl.BlockSpec((3, B, T), lambda i: (0, 0, i)),
                  pl.BlockSpec((3, B, P), lambda i: (0, 0, 0))],
        out_specs=pl.BlockSpec((B, T, KNB), lambda i: (0, i, 0)),
        out_shape=jax.ShapeDtypeStruct((B, S, KNB), jnp.int32),
        interpret=interpret,
    )(q_t, p_t)


_GCH = 128


def _make_gather(V, D, M):
    info = plsc.get_sparse_core_info()
    _NC, _NS = info.num_cores, info.num_subcores
    _NW = _NC * _NS
    per_w = M // _NW
    nch = per_w // _GCH
    assert per_w % _GCH == 0 and M % _NW == 0
    mesh = plsc.VectorSubcoreMesh(core_axis_name="c", subcore_axis_name="s")

    @functools.partial(
        pl.kernel,
        mesh=mesh,
        out_type=jax.ShapeDtypeStruct((M, D), F32),
        scratch_types=[
            pltpu.VMEM((_GCH,), jnp.int32),
            pltpu.VMEM((_GCH, D), F32),
            pltpu.SemaphoreType.DMA,
        ],
    )
    def k(idx_hbm, table_hbm, out_hbm, idx_v, rows_v, sem):
        wid = lax.axis_index("s") * _NC + lax.axis_index("c")
        base = wid * per_w

        def body(i, carry):
            off = base + i * _GCH
            pltpu.sync_copy(idx_hbm.at[pl.ds(off, _GCH)], idx_v)
            pltpu.async_copy(table_hbm.at[idx_v], rows_v, sem).wait()
            pltpu.sync_copy(rows_v, out_hbm.at[pl.ds(off, _GCH)])
            return carry

        lax.fori_loop(0, nch, body, 0)

    return k


def _conv_a(QB, C, n_ref, hx_ref, wa_ref):
    nv = n_ref[...][:, :C]
    hx = hx_ref[...][:, :C]
    sub = nv.reshape(QB, KNB, C) - hx[:, None, :]
    xb = jnp.broadcast_to(hx[:, None, :], (QB, KNB, C))
    feat = jnp.concatenate([_bf(sub).reshape(QB * KNB, C),
                            _bf(xb).reshape(QB * KNB, C)], axis=1)
    return jnp.dot(feat, _bf(wa_ref[...]), preferred_element_type=F32)


def _p1_body(QB, C, n_ref, hx_ref, wa_ref, st_ref):
    i = pl.program_id(0)
    y = _conv_a(QB, C, n_ref, hx_ref, wa_ref)
    contrib = _stats_contrib(y)

    @pl.when(i == 0)
    def _():
        st_ref[...] = jnp.zeros_like(st_ref)

    st_ref[...] += contrib


def _p2_body(QB, C, cnt, n_ref, hx_ref, wa_ref, st_ref, g_ref, b_ref, w_ref,
             o_ref, st2_ref):
    i = pl.program_id(0)
    scale, shift = _norm_consts(st_ref[...], cnt, g_ref[...], b_ref[...])
    y = _conv_a(QB, C, n_ref, hx_ref, wa_ref)
    yn = jnp.maximum(y * scale + shift, 0.0)
    y2 = _bdot(yn, w_ref[...])
    o_ref[...] = y2
    contrib = _stats_contrib(y2)

    @pl.when(i == 0)
    def _():
        st2_ref[...] = jnp.zeros_like(st2_ref)

    st2_ref[...] += contrib


def _pool_body(QB, CO, cnt, y_ref, st_ref, g_ref, b_ref, p_ref):
    scale, shift = _norm_consts(st_ref[...], cnt, g_ref[...], b_ref[...])
    yn = jnp.maximum(y_ref[...] * scale + shift, 0.0).reshape(QB, KNB, CO)
    p_ref[...] = jnp.max(yn, axis=1)


def _p3_final_body(QB, CO, cnt, y_ref, st_ref, g_ref, b_ref, wo_ref, y3_ref, st3_ref):
    i = pl.program_id(0)
    scale, shift = _norm_consts(st_ref[...], cnt, g_ref[...], b_ref[...])
    yn = jnp.maximum(y_ref[...] * scale + shift, 0.0).reshape(QB, KNB, CO)
    pooled = jnp.max(yn, axis=1)
    y3 = _bdot(pooled, wo_ref[...])
    y3_ref[...] = y3
    contrib = _stats_contrib(y3)

    @pl.when(i == 0)
    def _():
        st3_ref[...] = jnp.zeros_like(st3_ref)

    st3_ref[...] += contrib


def _p5_body(y_ref, st_ref, g_ref, b_ref, o_ref):
    scale, shift = _norm_consts(st_ref[...], float(B * S2), g_ref[...], b_ref[...])
    o_ref[...] = (y_ref[...] * scale + shift)[None]


def _stage_mlp(n_rows, C, nflat, hxflat, wat, ga, ba, wbt, interpret=False):
    CO = 2 * C
    RB = 4096
    QB = RB // KNB
    grid = (n_rows // RB,)
    cnt = float(n_rows)
    DW = nflat.shape[1]
    st_a = pl.pallas_call(
        functools.partial(_p1_body, QB, C),
        grid=grid,
        in_specs=[pl.BlockSpec((RB, DW), lambda i: (i, 0)),
                  pl.BlockSpec((QB, DW), lambda i: (i, 0)),
                  pl.BlockSpec((CO, CO), lambda i: (0, 0))],
        out_specs=pl.BlockSpec((8, CO), lambda i: (0, 0)),
        out_shape=jax.ShapeDtypeStruct((8, CO), F32),
        interpret=interpret,
    )(nflat, hxflat, wat)
    y2, st_b = pl.pallas_call(
        functools.partial(_p2_body, QB, C, cnt),
        grid=grid,
        in_specs=[pl.BlockSpec((RB, DW), lambda i: (i, 0)),
                  pl.BlockSpec((QB, DW), lambda i: (i, 0)),
                  pl.BlockSpec((CO, CO), lambda i: (0, 0)),
                  pl.BlockSpec((8, CO), lambda i: (0, 0)),
                  pl.BlockSpec((1, CO), lambda i: (0, 0)),
                  pl.BlockSpec((1, CO), lambda i: (0, 0)),
                  pl.BlockSpec((CO, CO), lambda i: (0, 0))],
        out_specs=[pl.BlockSpec((RB, CO), lambda i: (i, 0)),
                   pl.BlockSpec((8, CO), lambda i: (0, 0))],
        out_shape=[jax.ShapeDtypeStruct((n_rows, CO), F32),
                   jax.ShapeDtypeStruct((8, CO), F32)],
        interpret=interpret,
    )(nflat, hxflat, wat, st_a, ga, ba, wbt)
    return y2, st_b


def kernel(x, w_emb1, gamma_emb1, beta_emb1, w_emb2, gamma_emb2, beta_emb2,
           w_sg1_a, gamma_sg1_a, beta_sg1_a, w_sg1_b, gamma_sg1_b, beta_sg1_b,
           w_sg2_a, gamma_sg2_a, beta_sg2_a, w_sg2_b, gamma_sg2_b, beta_sg2_b,
           w_out, gamma_out, beta_out):
    f = lambda a: a.astype(F32)
    x = f(x)
    xp = jnp.zeros((B * N, 128), F32).at[:, :3].set(x.reshape(B * N, 3))
    w1p = jnp.zeros((128, C1), F32).at[:3, :].set(f(w_emb1).T)
    coords_t = jnp.transpose(x, (2, 0, 1))
    r1 = lambda a: f(a).reshape(1, -1)

    h1 = _embed(xp, w1p, r1(gamma_emb1), r1(beta_emb1), f(w_emb2).T,
                r1(gamma_emb2), r1(beta_emb2))

    idx1, sct1 = _fps(coords_t, N, S1)
    knn1 = _knn(sct1, coords_t, N, S1)

    hx1 = _make_gather(B * N, 128, B * S1)(idx1.reshape(-1), h1)
    n1 = _make_gather(B * N, 128, B * S1 * KNB)(knn1.reshape(-1), h1)

    y2s1, stb1 = _stage_mlp(B * S1 * KNB, C1, n1, hx1, f(w_sg1_a).T,
                            r1(gamma_sg1_a), r1(beta_sg1_a), f(w_sg1_b).T)
    RB = 4096
    QB = RB // KNB
    pooled1 = pl.pallas_call(
        functools.partial(_pool_body, QB, C2, float(B * S1 * KNB)),
        grid=(B * S1 * KNB // RB,),
        in_specs=[pl.BlockSpec((RB, C2), lambda i: (i, 0)),
                  pl.BlockSpec((8, C2), lambda i: (0, 0)),
                  pl.BlockSpec((1, C2), lambda i: (0, 0)),
                  pl.BlockSpec((1, C2), lambda i: (0, 0))],
        out_specs=pl.BlockSpec((QB, C2), lambda i: (i, 0)),
        out_shape=jax.ShapeDtypeStruct((B * S1, C2), F32),
    )(y2s1, stb1, r1(gamma_sg1_b), r1(beta_sg1_b))

    idx2, sct2 = _fps(sct1, S1, S2)
    knn2 = _knn(sct2, sct1, S1, S2)

    hx2 = _make_gather(B * S1, C2, B * S2)(idx2.reshape(-1), pooled1)
    n2 = _make_gather(B * S1, C2, B * S2 * KNB)(knn2.reshape(-1), pooled1)

    y2s2, stb2 = _stage_mlp(B * S2 * KNB, C2, n2, hx2, f(w_sg2_a).T,
                            r1(gamma_sg2_a), r1(beta_sg2_a), f(w_sg2_b).T)
    y3, st3 = pl.pallas_call(
        functools.partial(_p3_final_body, QB, C3, float(B * S2 * KNB)),
        grid=(B * S2 * KNB // RB,),
        in_specs=[pl.BlockSpec((RB, C3), lambda i: (i, 0)),
                  pl.BlockSpec((8, C3), lambda i: (0, 0)),
                  pl.BlockSpec((1, C3), lambda i: (0, 0)),
                  pl.BlockSpec((1, C3), lambda i: (0, 0)),
                  pl.BlockSpec((C3, C3), lambda i: (0, 0))],
        out_specs=[pl.BlockSpec((QB, C3), lambda i: (i, 0)),
                   pl.BlockSpec((8, C3), lambda i: (0, 0))],
        out_shape=[jax.ShapeDtypeStruct((B * S2, C3), F32),
                   jax.ShapeDtypeStruct((8, C3), F32)],
    )(y2s2, stb2, r1(gamma_sg2_b), r1(beta_sg2_b), f(w_out).T)
    hfin = pl.pallas_call(
        _p5_body,
        grid=(B,),
        in_specs=[pl.BlockSpec((S2, C3), lambda i: (i, 0)),
                  pl.BlockSpec((8, C3), lambda i: (0, 0)),
                  pl.BlockSpec((1, C3), lambda i: (0, 0)),
                  pl.BlockSpec((1, C3), lambda i: (0, 0))],
        out_specs=pl.BlockSpec((1, S2, C3), lambda i: (i, 0, 0)),
        out_shape=jax.ShapeDtypeStruct((B, S2, C3), F32),
    )(y3, st3, r1(gamma_out), r1(beta_out))

    h_out = jnp.transpose(hfin, (0, 2, 1))
    coords_out = jnp.transpose(sct2, (1, 2, 0))
    return h_out, coords_out


# speedup vs baseline: 12.2781x; 1.0451x over previous
import functools

import jax
import jax.numpy as jnp
from jax import lax
from jax.experimental import pallas as pl
from jax.experimental.pallas import tpu as pltpu
from jax.experimental.pallas import tpu_sc as plsc

B = 8
N = 4096
KNB = 32
S1 = 2048
S2 = 1024
C1 = 64
C2 = 128
C3 = 256
EPS = 1e-5
BIGF = 3.0e38

F32 = jnp.float32


def _norm_consts(st, cnt, g, b):
    m = st[0:1, :] / cnt
    v = st[1:2, :] / cnt - m * m
    scale = g / jnp.sqrt(v + EPS)
    shift = b - m * scale
    return scale, shift


def _bf(a):
    return a.astype(jnp.bfloat16)


def _bdot(a, b):
    return jnp.dot(_bf(a), _bf(b), preferred_element_type=F32)


def _stats_contrib(y):
    s = jnp.sum(y, axis=0, keepdims=True)
    s2 = jnp.sum(y * y, axis=0, keepdims=True)
    pad = jnp.zeros((6, y.shape[-1]), F32)
    return jnp.concatenate([s, s2, pad], axis=0)


def _e1_body(x_ref, w_ref, y_ref, st_ref):
    i = pl.program_id(0)
    y = _bdot(x_ref[...], w_ref[...])
    y_ref[...] = y
    contrib = _stats_contrib(y)

    @pl.when(i == 0)
    def _():
        st_ref[...] = jnp.zeros_like(st_ref)

    st_ref[...] += contrib


def _mlp_mid_body(cnt, y_ref, st_ref, g_ref, b_ref, w_ref, o_ref, st2_ref):
    i = pl.program_id(0)
    scale, shift = _norm_consts(st_ref[...], cnt, g_ref[...], b_ref[...])
    yn = jnp.maximum(y_ref[...] * scale + shift, 0.0)
    y2 = _bdot(yn, w_ref[...])
    o_ref[...] = y2
    contrib = _stats_contrib(y2)

    @pl.when(i == 0)
    def _():
        st2_ref[...] = jnp.zeros_like(st2_ref)

    st2_ref[...] += contrib


def _hout_body(cnt, y_ref, st_ref, g_ref, b_ref, h_ref):
    scale, shift = _norm_consts(st_ref[...], cnt, g_ref[...], b_ref[...])
    yn = jnp.maximum(y_ref[...] * scale + shift, 0.0)
    h_ref[...] = jnp.concatenate(
        [yn, jnp.zeros((yn.shape[0], 128 - yn.shape[1]), F32)], axis=1)


def _embed(xp, w1p, g1, b1, w2t, g2, b2, interpret=False):
    R = B * N
    RB = 2048
    grid = (R // RB,)
    y1, st1 = pl.pallas_call(
        _e1_body,
        grid=grid,
        in_specs=[pl.BlockSpec((RB, 128), lambda i: (i, 0)),
                  pl.BlockSpec((128, C1), lambda i: (0, 0))],
        out_specs=[pl.BlockSpec((RB, C1), lambda i: (i, 0)),
                   pl.BlockSpec((8, C1), lambda i: (0, 0))],
        out_shape=[jax.ShapeDtypeStruct((R, C1), F32),
                   jax.ShapeDtypeStruct((8, C1), F32)],
        interpret=interpret,
    )(xp, w1p)
    y2, st2 = pl.pallas_call(
        functools.partial(_mlp_mid_body, float(R)),
        grid=grid,
        in_specs=[pl.BlockSpec((RB, C1), lambda i: (i, 0)),
                  pl.BlockSpec((8, C1), lambda i: (0, 0)),
                  pl.BlockSpec((1, C1), lambda i: (0, 0)),
                  pl.BlockSpec((1, C1), lambda i: (0, 0)),
                  pl.BlockSpec((C1, C1), lambda i: (0, 0))],
        out_specs=[pl.BlockSpec((RB, C1), lambda i: (i, 0)),
                   pl.BlockSpec((8, C1), lambda i: (0, 0))],
        out_shape=[jax.ShapeDtypeStruct((R, C1), F32),
                   jax.ShapeDtypeStruct((8, C1), F32)],
        interpret=interpret,
    )(y1, st1, g1, b1, w2t)
    h = pl.pallas_call(
        functools.partial(_hout_body, float(R)),
        grid=grid,
        in_specs=[pl.BlockSpec((RB, C1), lambda i: (i, 0)),
                  pl.BlockSpec((8, C1), lambda i: (0, 0)),
                  pl.BlockSpec((1, C1), lambda i: (0, 0)),
                  pl.BlockSpec((1, C1), lambda i: (0, 0))],
        out_specs=pl.BlockSpec((RB, 128), lambda i: (i, 0)),
        out_shape=jax.ShapeDtypeStruct((R, 128), F32),
        interpret=interpret,
    )(y2, st2, g2, b2)
    return h


def _fps_body(P, S, c_ref, idx_ref, sc_ref):
    px = c_ref[0]
    py = c_ref[1]
    pz = c_ref[2]
    boff = lax.broadcasted_iota(jnp.int32, (B, 1), 0) * P
    lane = lax.broadcasted_iota(jnp.int32, (B, P), 1)
    slane = lax.broadcasted_iota(jnp.int32, (B, S), 1)
    q0x = px[:, 0:1]
    q0y = py[:, 0:1]
    q0z = pz[:, 0:1]
    idx_acc = jnp.broadcast_to(boff, (B, S))
    qx_acc = jnp.broadcast_to(q0x, (B, S))
    qy_acc = jnp.broadcast_to(q0y, (B, S))
    qz_acc = jnp.broadcast_to(q0z, (B, S))
    dx = px - q0x
    dy = py - q0y
    dz = pz - q0z
    mind = (dx * dx + dy * dy) + dz * dz

    def body(i, carry):
        mind, idx_acc, qx_acc, qy_acc, qz_acc = carry
        mx = jnp.max(mind, axis=1, keepdims=True)
        am = jnp.min(jnp.where(mind == mx, lane, P), axis=1, keepdims=True)
        onehot = lane == am
        qx = jnp.sum(jnp.where(onehot, px, 0.0), axis=1, keepdims=True)
        qy = jnp.sum(jnp.where(onehot, py, 0.0), axis=1, keepdims=True)
        qz = jnp.sum(jnp.where(onehot, pz, 0.0), axis=1, keepdims=True)
        sel = slane == i
        idx_acc = jnp.where(sel, am + boff, idx_acc)
        qx_acc = jnp.where(sel, qx, qx_acc)
        qy_acc = jnp.where(sel, qy, qy_acc)
        qz_acc = jnp.where(sel, qz, qz_acc)
        ddx = px - qx
        ddy = py - qy
        ddz = pz - qz
        d = (ddx * ddx + ddz * ddz) + ddy * ddy
        return (jnp.minimum(mind, d), idx_acc, qx_acc, qy_acc, qz_acc)

    carry = (mind, idx_acc, qx_acc, qy_acc, qz_acc)
    _, idx_acc, qx_acc, qy_acc, qz_acc = lax.fori_loop(1, S, body, carry)
    idx_ref[...] = idx_acc
    sc_ref[0] = qx_acc
    sc_ref[1] = qy_acc
    sc_ref[2] = qz_acc


def _fps(coords_t, P, S, interpret=False):
    return pl.pallas_call(
        functools.partial(_fps_body, P, S),
        in_specs=[pl.BlockSpec((3, B, P), lambda: (0, 0, 0))],
        out_specs=[pl.BlockSpec((B, S), lambda: (0, 0)),
                   pl.BlockSpec((3, B, S), lambda: (0, 0, 0))],
        out_shape=[jax.ShapeDtypeStruct((B, S), jnp.int32),
                   jax.ShapeDtypeStruct((3, B, S), F32)],
        interpret=interpret,
    )(coords_t)


def _knn_body(P, T, q_ref, p_ref, o_ref):
    qx = q_ref[0][:, :, None]
    qy = q_ref[1][:, :, None]
    qz = q_ref[2][:, :, None]
    px = p_ref[0][:, None, :]
    py = p_ref[1][:, None, :]
    pz = p_ref[2][:, None, :]
    q2 = (qx * qx + qy * qy) + qz * qz
    p2 = (px * px + py * py) + pz * pz
    bf = lambda a: a.astype(jnp.bfloat16).astype(F32)
    qp = bf(qx) * bf(px) + bf(qy) * bf(py) + bf(qz) * bf(pz)
    d = (q2 - 2.0 * qp) + p2
    lane = lax.broadcasted_iota(jnp.int32, (B, T, P), 2)
    boff = lax.broadcasted_iota(jnp.int32, (B, 1, 1), 0) * P
    picks = []
    for _ in range(KNB):
        am = jnp.argmin(d, axis=2).astype(jnp.int32)[:, :, None]
        picks.append(am)
        d = jnp.where(lane == am, BIGF, d)
    o_ref[...] = jnp.concatenate(picks, axis=2) + boff


def _knn(q_t, p_t, P, S, interpret=False):
    T = 128
    return pl.pallas_call(
        functools.partial(_knn_body, P, T),
        grid=(S // T,),
        in_specs=[pl.BlockSpec((3, B, T), lambda i: (0, 0, i)),
                  pl.BlockSpec((3, B, P), lambda i: (0, 0, 0))],
        out_specs=pl.BlockSpec((B, T, KNB), lambda i: (0, i, 0)),
        out_shape=jax.ShapeDtypeStruct((B, S, KNB), jnp.int32),
        interpret=interpret,
    )(q_t, p_t)


_GCH = 128


def _make_gather(V, D, M):
    info = plsc.get_sparse_core_info()
    _NC, _NS = info.num_cores, info.num_subcores
    _NW = _NC * _NS
    per_w = M // _NW
    nch = per_w // _GCH
    assert per_w % _GCH == 0 and M % _NW == 0
    mesh = plsc.VectorSubcoreMesh(core_axis_name="c", subcore_axis_name="s")

    @functools.partial(
        pl.kernel,
        mesh=mesh,
        out_type=jax.ShapeDtypeStruct((M, D), F32),
        scratch_types=[
            pltpu.VMEM((_GCH,), jnp.int32),
            pltpu.VMEM((_GCH, D), F32),
            pltpu.SemaphoreType.DMA,
        ],
    )
    def k(idx_hbm, table_hbm, out_hbm, idx_v, rows_v, sem):
        wid = lax.axis_index("s") * _NC + lax.axis_index("c")
        base = wid * per_w

        def body(i, carry):
            off = base + i * _GCH
            pltpu.sync_copy(idx_hbm.at[pl.ds(off, _GCH)], idx_v)
            pltpu.async_copy(table_hbm.at[idx_v], rows_v, sem).wait()
            pltpu.sync_copy(rows_v, out_hbm.at[pl.ds(off, _GCH)])
            return carry

        lax.fori_loop(0, nch, body, 0)

    return k


def _conv_a(QB, C, n_ref, hx_ref, wa_ref):
    nv = n_ref[...][:, :C]
    hx = hx_ref[...][:, :C]
    sub = nv.reshape(QB, KNB, C) - hx[:, None, :]
    xb = jnp.broadcast_to(hx[:, None, :], (QB, KNB, C))
    feat = jnp.concatenate([_bf(sub).reshape(QB * KNB, C),
                            _bf(xb).reshape(QB * KNB, C)], axis=1)
    return jnp.dot(feat, _bf(wa_ref[...]), preferred_element_type=F32)


def _p1_body(QB, C, n_ref, hx_ref, wa_ref, st_ref):
    i = pl.program_id(0)
    y = _conv_a(QB, C, n_ref, hx_ref, wa_ref)
    contrib = _stats_contrib(y)

    @pl.when(i == 0)
    def _():
        st_ref[...] = jnp.zeros_like(st_ref)

    st_ref[...] += contrib


def _p2_body(QB, C, cnt, n_ref, hx_ref, wa_ref, st_ref, g_ref, b_ref, w_ref,
             o_ref, st2_ref):
    i = pl.program_id(0)
    scale, shift = _norm_consts(st_ref[...], cnt, g_ref[...], b_ref[...])
    y = _conv_a(QB, C, n_ref, hx_ref, wa_ref)
    yn = jnp.maximum(y * scale + shift, 0.0)
    y2 = _bdot(yn, w_ref[...])
    o_ref[...] = y2
    contrib = _stats_contrib(y2)

    @pl.when(i == 0)
    def _():
        st2_ref[...] = jnp.zeros_like(st2_ref)

    st2_ref[...] += contrib


def _pool_body(QB, CO, cnt, y_ref, st_ref, g_ref, b_ref, p_ref):
    scale, shift = _norm_consts(st_ref[...], cnt, g_ref[...], b_ref[...])
    yn = jnp.maximum(y_ref[...] * scale + shift, 0.0).reshape(QB, KNB, CO)
    p_ref[...] = jnp.max(yn, axis=1)


def _p3_final_body(QB, CO, cnt, y_ref, st_ref, g_ref, b_ref, wo_ref, y3_ref, st3_ref):
    i = pl.program_id(0)
    scale, shift = _norm_consts(st_ref[...], cnt, g_ref[...], b_ref[...])
    yn = jnp.maximum(y_ref[...] * scale + shift, 0.0).reshape(QB, KNB, CO)
    pooled = jnp.max(yn, axis=1)
    y3 = _bdot(pooled, wo_ref[...])
    y3_ref[...] = y3
    contrib = _stats_contrib(y3)

    @pl.when(i == 0)
    def _():
        st3_ref[...] = jnp.zeros_like(st3_ref)

    st3_ref[...] += contrib


def _p5_body(y_ref, st_ref, g_ref, b_ref, o_ref):
    scale, shift = _norm_consts(st_ref[...], float(B * S2), g_ref[...], b_ref[...])
    o_ref[...] = (y_ref[...] * scale + shift)[None]


def _stage_mlp(n_rows, C, nflat, hxflat, wat, ga, ba, wbt, interpret=False):
    CO = 2 * C
    RB = 4096
    QB = RB // KNB
    grid = (n_rows // RB,)
    cnt = float(n_rows)
    DW = nflat.shape[1]
    st_a = pl.pallas_call(
        functools.partial(_p1_body, QB, C),
        grid=grid,
        in_specs=[pl.BlockSpec((RB, DW), lambda i: (i, 0)),
                  pl.BlockSpec((QB, DW), lambda i: (i, 0)),
                  pl.BlockSpec((CO, CO), lambda i: (0, 0))],
        out_specs=pl.BlockSpec((8, CO), lambda i: (0, 0)),
        out_shape=jax.ShapeDtypeStruct((8, CO), F32),
        interpret=interpret,
    )(nflat, hxflat, wat)
    y2, st_b = pl.pallas_call(
        functools.partial(_p2_body, QB, C, cnt),
        grid=grid,
        in_specs=[pl.BlockSpec((RB, DW), lambda i: (i, 0)),
                  pl.BlockSpec((QB, DW), lambda i: (i, 0)),
                  pl.BlockSpec((CO, CO), lambda i: (0, 0)),
                  pl.BlockSpec((8, CO), lambda i: (0, 0)),
                  pl.BlockSpec((1, CO), lambda i: (0, 0)),
                  pl.BlockSpec((1, CO), lambda i: (0, 0)),
                  pl.BlockSpec((CO, CO), lambda i: (0, 0))],
        out_specs=[pl.BlockSpec((RB, CO), lambda i: (i, 0)),
                   pl.BlockSpec((8, CO), lambda i: (0, 0))],
        out_shape=[jax.ShapeDtypeStruct((n_rows, CO), F32),
                   jax.ShapeDtypeStruct((8, CO), F32)],
        interpret=interpret,
    )(nflat, hxflat, wat, st_a, ga, ba, wbt)
    return y2, st_b


def kernel(x, w_emb1, gamma_emb1, beta_emb1, w_emb2, gamma_emb2, beta_emb2,
           w_sg1_a, gamma_sg1_a, beta_sg1_a, w_sg1_b, gamma_sg1_b, beta_sg1_b,
           w_sg2_a, gamma_sg2_a, beta_sg2_a, w_sg2_b, gamma_sg2_b, beta_sg2_b,
           w_out, gamma_out, beta_out):
    f = lambda a: a.astype(F32)
    x = f(x)
    xp = jnp.zeros((B * N, 128), F32).at[:, :3].set(x.reshape(B * N, 3))
    w1p = jnp.zeros((128, C1), F32).at[:3, :].set(f(w_emb1).T)
    coords_t = jnp.transpose(x, (2, 0, 1))
    r1 = lambda a: f(a).reshape(1, -1)

    h1 = _embed(xp, w1p, r1(gamma_emb1), r1(beta_emb1), f(w_emb2).T,
                r1(gamma_emb2), r1(beta_emb2))

    idx1, sct1 = _fps(coords_t, N, S1)
    knn1 = _knn(sct1, coords_t, N, S1)

    hx1 = _make_gather(B * N, 128, B * S1)(idx1.reshape(-1), h1)
    n1 = _make_gather(B * N, 128, B * S1 * KNB)(knn1.reshape(-1), h1)

    y2s1, stb1 = _stage_mlp(B * S1 * KNB, C1, n1, hx1, f(w_sg1_a).T,
                            r1(gamma_sg1_a), r1(beta_sg1_a), f(w_sg1_b).T)
    RB = 4096
    QB = RB // KNB
    pooled1 = pl.pallas_call(
        functools.partial(_pool_body, QB, C2, float(B * S1 * KNB)),
        grid=(B * S1 * KNB // RB,),
        in_specs=[pl.BlockSpec((RB, C2), lambda i: (i, 0)),
                  pl.BlockSpec((8, C2), lambda i: (0, 0)),
                  pl.BlockSpec((1, C2), lambda i: (0, 0)),
                  pl.BlockSpec((1, C2), lambda i: (0, 0))],
        out_specs=pl.BlockSpec((QB, C2), lambda i: (i, 0)),
        out_shape=jax.ShapeDtypeStruct((B * S1, C2), F32),
    )(y2s1, stb1, r1(gamma_sg1_b), r1(beta_sg1_b))

    idx2, sct2 = _fps(sct1, S1, S2)
    knn2 = _knn(sct2, sct1, S1, S2)

    hx2 = _make_gather(B * S1, C2, B * S2)(idx2.reshape(-1), pooled1)
    n2 = _make_gather(B * S1, C2, B * S2 * KNB)(knn2.reshape(-1), pooled1)

    y2s2, stb2 = _stage_mlp(B * S2 * KNB, C2, n2, hx2, f(w_sg2_a).T,
                            r1(gamma_sg2_a), r1(beta_sg2_a), f(w_sg2_b).T)
    y3, st3 = pl.pallas_call(
        functools.partial(_p3_final_body, QB, C3, float(B * S2 * KNB)),
        grid=(B * S2 * KNB // RB,),
        in_specs=[pl.BlockSpec((RB, C3), lambda i: (i, 0)),
                  pl.BlockSpec((8, C3), lambda i: (0, 0)),
                  pl.BlockSpec((1, C3), lambda i: (0, 0)),
                  pl.BlockSpec((1, C3), lambda i: (0, 0)),
                  pl.BlockSpec((C3, C3), lambda i: (0, 0))],
        out_specs=[pl.BlockSpec((QB, C3), lambda i: (i, 0)),
                   pl.BlockSpec((8, C3), lambda i: (0, 0))],
        out_shape=[jax.ShapeDtypeStruct((B * S2, C3), F32),
                   jax.ShapeDtypeStruct((8, C3), F32)],
    )(y2s2, stb2, r1(gamma_sg2_b), r1(beta_sg2_b), f(w_out).T)
    hfin = pl.pallas_call(
        _p5_body,
        grid=(B,),
        in_specs=[pl.BlockSpec((S2, C3), lambda i: (i, 0)),
                  pl.BlockSpec((8, C3), lambda i: (0, 0)),
                  pl.BlockSpec((1, C3), lambda i: (0, 0)),
                  pl.BlockSpec((1, C3), lambda i: (0, 0))],
        out_specs=pl.BlockSpec((1, S2, C3), lambda i: (i, 0, 0)),
        out_shape=jax.ShapeDtypeStruct((B, S2, C3), F32),
    )(y3, st3, r1(gamma_out), r1(beta_out))

    h_out = jnp.transpose(hfin, (0, 2, 1))
    coords_out = jnp.transpose(sct2, (1, 2, 0))
    return h_out, coords_out
